# baseline (speedup 1.0000x reference)
import functools

import jax
import jax.numpy as jnp
from jax import lax
from jax.experimental import pallas as pl
from jax.experimental.pallas import tpu as pltpu
from jax.experimental.pallas import tpu_sc as plsc

N = 100000
D = 128
NG = 64
L = 16
NWORK = 32

R_TC = 10000
S = 30000
NBLK_TC = (N - S) // R_TC
OFF_TC = S // R_TC

G = S // L
W = 20
GPW = 60
NCH = -(-GPW // W)
ROWS_W = W * L

_mesh = plsc.VectorSubcoreMesh(core_axis_name="c", subcore_axis_name="s")


@functools.partial(
    pl.kernel,
    mesh=_mesh,
    out_type=jax.ShapeDtypeStruct((NWORK, NG * L), jnp.float32),
    scratch_types=[
        pltpu.VMEM((2, ROWS_W, D), jnp.float32),
        pltpu.VMEM((ROWS_W,), jnp.int32),
        pltpu.VMEM((ROWS_W,), jnp.int32),
        pltpu.VMEM((NG * L,), jnp.float32),
        pltpu.SemaphoreType.DMA,
        pltpu.SemaphoreType.DMA,
        pltpu.SemaphoreType.DMA,
        pltpu.SemaphoreType.DMA,
    ],
)
def _sc_partials(x_hbm, b_hbm, out_hbm, xbuf, bbufA, bbufB, bucket,
                 sx0, sx1, sb0, sb1):
    wid = lax.axis_index("s") * 2 + lax.axis_index("c")
    wstart = wid * GPW
    wend = jnp.minimum(wstart + GPW, G)

    for i in range(NG):
        bucket[pl.ds(i * L, L)] = jnp.zeros((L,), jnp.float32)

    semx = [sx0, sx1]
    semb = [sb0, sb1]
    bbufs = [bbufA, bbufB]

    def window_start(c):
        return jnp.clip(wstart + c * W, 0, G - W)

    def start_dma(c):
        r0 = window_start(c) * L
        cpx = pltpu.async_copy(
            x_hbm.at[pl.ds(r0, ROWS_W)], xbuf.at[c % 2], semx[c % 2])
        cpb = pltpu.async_copy(
            b_hbm.at[pl.ds(r0, ROWS_W)], bbufs[c % 2], semb[c % 2])
        return cpx, cpb

    inflight = start_dma(0)
    for c in range(NCH):
        cpx, cpb = inflight
        if c + 1 < NCH:
            inflight = start_dma(c + 1)
        cpx.wait()
        cpb.wait()
        ws = window_start(c)
        cg0 = wstart + c * W
        buf = c % 2

        def group_body(j, _, ws=ws, cg0=cg0, buf=buf):
            gid = ws + j
            b_vec = bbufs[buf][pl.ds(j * L, L)]
            uniform = b_vec[0] == b_vec[L - 1]
            valid = (gid >= cg0) & (gid < wend)

            @pl.when(valid & uniform)
            def _():
                accs = [jnp.zeros((L,), jnp.float32) for _ in range(4)]
                for r in range(L):
                    row = j * L + r
                    for cc in range(D // L):
                        v = xbuf[buf, row, pl.ds(cc * L, L)]
                        accs[cc % 4] = accs[cc % 4] + v * v
                acc = (accs[0] + accs[1]) + (accs[2] + accs[3])
                base = b_vec[0] * L
                bucket[pl.ds(base, L)] = bucket[pl.ds(base, L)] + acc

            @pl.when(valid & jnp.logical_not(uniform))
            def _():
                for r in range(L):
                    row = j * L + r
                    racc = jnp.zeros((L,), jnp.float32)
                    for cc in range(D // L):
                        v = xbuf[buf, row, pl.ds(cc * L, L)]
                        racc = racc + v * v
                    base = b_vec[r] * L
                    bucket[pl.ds(base, L)] = bucket[pl.ds(base, L)] + racc

            return 0

        lax.fori_loop(0, W, group_body, 0)

    for i in range(NG):
        bucket[pl.ds(i * L, L)] = bucket[pl.ds(i * L, L)] * 0.5
    pltpu.sync_copy(bucket, out_hbm.at[wid])


def _tc_body(b_ref, x_ref, out_ref):
    i = pl.program_id(0)

    @pl.when(i == 0)
    def _():
        out_ref[...] = jnp.zeros_like(out_ref)

    x = x_ref[...]
    b = b_ref[0, 0, :]
    onehot = (b[None, :] == lax.iota(jnp.int32, NG)[:, None]).astype(jnp.float32)
    e = jnp.dot(onehot, x * x, preferred_element_type=jnp.float32)
    out_ref[...] += 0.5 * jnp.sum(e, axis=1)[None, :]


_tc_reduce = pl.pallas_call(
    _tc_body,
    grid=(NBLK_TC,),
    in_specs=[
        pl.BlockSpec((1, 1, R_TC), lambda i: (i + OFF_TC, 0, 0)),
        pl.BlockSpec((R_TC, D), lambda i: (i + OFF_TC, 0)),
    ],
    out_specs=pl.BlockSpec((1, NG), lambda i: (0, 0)),
    out_shape=jax.ShapeDtypeStruct((1, NG), jnp.float32),
)


def kernel(X, batch, num_graphs):
    del num_graphs
    b32 = batch.astype(jnp.int32)
    sc_part = _sc_partials(X, b32)
    tc_part = _tc_reduce(b32.reshape(N // R_TC, 1, R_TC), X)
    return tc_part[0] + jnp.sum(sc_part.reshape(NWORK, NG, L), axis=(0, 2))


# speedup vs baseline: 1.0637x; 1.0637x over previous
import functools

import jax
import jax.numpy as jnp
from jax import lax
from jax.experimental import pallas as pl
from jax.experimental.pallas import tpu as pltpu
from jax.experimental.pallas import tpu_sc as plsc

N = 100000
D = 128
NG = 64
L = 16
NWORK = 32

R_TC = 20000
S = 20000
NBLK_TC = (N - S) // R_TC
OFF_TC = S // R_TC

G = S // L
W = 20
GPW = 40
NCH = -(-GPW // W)
ROWS_W = W * L

_mesh = plsc.VectorSubcoreMesh(core_axis_name="c", subcore_axis_name="s")


@functools.partial(
    pl.kernel,
    mesh=_mesh,
    out_type=jax.ShapeDtypeStruct((NWORK, NG * L), jnp.float32),
    scratch_types=[
        pltpu.VMEM((2, ROWS_W, D), jnp.float32),
        pltpu.VMEM((ROWS_W,), jnp.int32),
        pltpu.VMEM((ROWS_W,), jnp.int32),
        pltpu.VMEM((NG * L,), jnp.float32),
        pltpu.SemaphoreType.DMA,
        pltpu.SemaphoreType.DMA,
        pltpu.SemaphoreType.DMA,
        pltpu.SemaphoreType.DMA,
    ],
)
def _sc_partials(x_hbm, b_hbm, out_hbm, xbuf, bbufA, bbufB, bucket,
                 sx0, sx1, sb0, sb1):
    wid = lax.axis_index("s") * 2 + lax.axis_index("c")
    wstart = wid * GPW
    wend = jnp.minimum(wstart + GPW, G)

    for i in range(NG):
        bucket[pl.ds(i * L, L)] = jnp.zeros((L,), jnp.float32)

    semx = [sx0, sx1]
    semb = [sb0, sb1]
    bbufs = [bbufA, bbufB]

    def window_start(c):
        return jnp.clip(wstart + c * W, 0, G - W)

    def start_dma(c):
        r0 = window_start(c) * L
        cpx = pltpu.async_copy(
            x_hbm.at[pl.ds(r0, ROWS_W)], xbuf.at[c % 2], semx[c % 2])
        cpb = pltpu.async_copy(
            b_hbm.at[pl.ds(r0, ROWS_W)], bbufs[c % 2], semb[c % 2])
        return cpx, cpb

    inflight = start_dma(0)
    for c in range(NCH):
        cpx, cpb = inflight
        if c + 1 < NCH:
            inflight = start_dma(c + 1)
        cpx.wait()
        cpb.wait()
        ws = window_start(c)
        cg0 = wstart + c * W
        buf = c % 2

        def group_body(j, _, ws=ws, cg0=cg0, buf=buf):
            gid = ws + j
            b_vec = bbufs[buf][pl.ds(j * L, L)]
            uniform = b_vec[0] == b_vec[L - 1]
            valid = (gid >= cg0) & (gid < wend)

            @pl.when(valid & uniform)
            def _():
                accs = [jnp.zeros((L,), jnp.float32) for _ in range(4)]
                for r in range(L):
                    row = j * L + r
                    for cc in range(D // L):
                        v = xbuf[buf, row, pl.ds(cc * L, L)]
                        accs[cc % 4] = accs[cc % 4] + v * v
                acc = (accs[0] + accs[1]) + (accs[2] + accs[3])
                base = b_vec[0] * L
                bucket[pl.ds(base, L)] = bucket[pl.ds(base, L)] + acc

            @pl.when(valid & jnp.logical_not(uniform))
            def _():
                for r in range(L):
                    row = j * L + r
                    racc = jnp.zeros((L,), jnp.float32)
                    for cc in range(D // L):
                        v = xbuf[buf, row, pl.ds(cc * L, L)]
                        racc = racc + v * v
                    base = b_vec[r] * L
                    bucket[pl.ds(base, L)] = bucket[pl.ds(base, L)] + racc

            return 0

        lax.fori_loop(0, W, group_body, 0)

    for i in range(NG):
        bucket[pl.ds(i * L, L)] = bucket[pl.ds(i * L, L)] * 0.5
    pltpu.sync_copy(bucket, out_hbm.at[wid])


def _tc_body(b_ref, x_ref, out_ref):
    i = pl.program_id(0)

    @pl.when(i == 0)
    def _():
        out_ref[...] = jnp.zeros_like(out_ref)

    x = x_ref[...]
    b = b_ref[0, 0, :]
    onehot = (b[None, :] == lax.iota(jnp.int32, NG)[:, None]).astype(jnp.float32)
    e = jnp.dot(onehot, x * x, preferred_element_type=jnp.float32)
    out_ref[...] += 0.5 * jnp.sum(e, axis=1)[None, :]


_tc_reduce = pl.pallas_call(
    _tc_body,
    grid=(NBLK_TC,),
    in_specs=[
        pl.BlockSpec((1, 1, R_TC), lambda i: (i + OFF_TC, 0, 0)),
        pl.BlockSpec((R_TC, D), lambda i: (i + OFF_TC, 0)),
    ],
    out_specs=pl.BlockSpec((1, NG), lambda i: (0, 0)),
    out_shape=jax.ShapeDtypeStruct((1, NG), jnp.float32),
)


def kernel(X, batch, num_graphs):
    del num_graphs
    b32 = batch.astype(jnp.int32)
    sc_part = _sc_partials(X, b32)
    tc_part = _tc_reduce(b32.reshape(N // R_TC, 1, R_TC), X)
    return tc_part[0] + jnp.sum(sc_part.reshape(NWORK, NG, L), axis=(0, 2))
